# baseline (speedup 1.0000x reference)
import functools

import jax
import jax.numpy as jnp
from jax import lax
from jax.experimental import pallas as pl
from jax.experimental.pallas import tpu as pltpu
from jax.experimental.pallas import tpu_sc as plsc

_LANES = 128


def _sc_gather(table, idx, chunk):
    _, d = table.shape
    b = idx.shape[0]
    info = plsc.get_sparse_core_info()
    nc = info.num_cores
    nw = nc * info.num_subcores
    b_per_w = b // nw
    assert b_per_w * nw == b and b_per_w % chunk == 0 and chunk % 8 == 0
    n_chunks = b_per_w // chunk
    nbuf = 8
    lag = 3
    n_groups = (n_chunks + lag + nbuf - 1) // nbuf
    mesh = plsc.VectorSubcoreMesh(core_axis_name="c", subcore_axis_name="s")

    @functools.partial(
        pl.kernel,
        mesh=mesh,
        out_type=jax.ShapeDtypeStruct((b, d), table.dtype),
        scratch_types=(
            [pltpu.VMEM((b_per_w,), jnp.int32)]
            + [pltpu.VMEM((chunk, d), table.dtype)] * nbuf
            + [pltpu.SemaphoreType.DMA] * (2 * nbuf)
        ),
    )
    def gather_kernel(table_hbm, idx_hbm, out_hbm, idx_v, *bs):
        bufs = bs[:nbuf]
        gsems = bs[nbuf:2 * nbuf]
        wsems = bs[2 * nbuf:]
        wid = lax.axis_index("s") * nc + lax.axis_index("c")
        base = wid * b_per_w
        pltpu.sync_copy(idx_hbm.at[pl.ds(base, b_per_w)], idx_v)

        def gat(ci, k):
            pltpu.async_copy(
                table_hbm.at[idx_v.at[pl.ds(ci * chunk, chunk)]],
                bufs[k], gsems[k])

        def wrb(ci, k):
            pltpu.async_copy(
                bufs[k], out_hbm.at[pl.ds(base + ci * chunk, chunk)],
                wsems[k])

        def wait_g(k):
            pltpu.make_async_copy(
                table_hbm.at[idx_v.at[pl.ds(0, chunk)]],
                bufs[k], gsems[k]).wait()

        def wait_w(k):
            pltpu.make_async_copy(
                bufs[k], out_hbm.at[pl.ds(base, chunk)], wsems[k]).wait()

        def body(j, carry):
            c0 = nbuf * j
            for k in range(nbuf):
                c = c0 + k

                @pl.when(jnp.logical_and(c >= nbuf, c < n_chunks))
                def _():
                    wait_w(k)

                @pl.when(c < n_chunks)
                def _():
                    gat(c, k)

                c2 = c - lag
                k2 = (k + nbuf - lag) % nbuf

                @pl.when(jnp.logical_and(c2 >= 0, c2 < n_chunks))
                def _():
                    wait_g(k2)
                    wrb(c2, k2)

            return carry

        lax.fori_loop(0, n_groups, body, 0)
        for k in range(nbuf):
            wait_w(k)

    return gather_kernel(table, idx)


def _sage_layer(feats, m_t, p, gru, relu, block):
    n = feats.shape[0]
    deg, _, mw = m_t.shape
    d_in, d_out = p["W_self"].shape
    f32 = jnp.float32

    def sig(v):
        return 0.5 * jnp.tanh(0.5 * v) + 0.5

    fuse = mw == 2 * d_in

    def body(*refs):
        if gru is not None:
            (f_ref, m_ref, wx_ref, wh_ref, bl_ref, ws_ref, bs_ref, wn_ref,
             hprev_ref, wih_ref, whh_ref, bih_ref, bhh_ref, out_ref) = refs
        else:
            (f_ref, m_ref, wx_ref, wh_ref, bl_ref, ws_ref, bs_ref, wn_ref,
             out_ref) = refs
        bf = jnp.bfloat16
        f = f_ref[...][:, :d_in]
        bl = bl_ref[...].astype(bf)
        h = jnp.zeros((block, d_in), bf)
        c = jnp.zeros((block, d_in), bf)
        def lstm_update(T, c):
            ti = T[:, :d_in]
            tf = T[:, d_in:2 * d_in]
            tg = T[:, 2 * d_in:3 * d_in]
            to = T[:, 3 * d_in:]
            c = 0.5 * ((tf + 1.0) * c + (ti + 1.0) * tg)
            h = (0.5 * to + 0.5) * jnp.tanh(c)
            return h, c

        if fuse:
            wcat = jnp.concatenate(
                [wx_ref[...], wh_ref[...]], axis=0).astype(bf)
            zero_low = jnp.zeros((block, d_in), bf)
            hu = jnp.zeros((block, mw), bf)
            for t in range(deg):
                u = m_ref[t].astype(bf) + hu
                z = jnp.dot(u, wcat, preferred_element_type=f32
                            ).astype(bf) + bl
                h, c = lstm_update(jnp.tanh(z), c)
                hu = jnp.concatenate([zero_low, h], axis=1)
        else:
            wx = wx_ref[...].astype(bf)
            wh = wh_ref[...].astype(bf)
            for t in range(deg):
                z = (jnp.dot(m_ref[t][:, :d_in].astype(bf), wx,
                             preferred_element_type=f32)
                     + jnp.dot(h, wh, preferred_element_type=f32)
                     ).astype(bf) + bl
                h, c = lstm_update(jnp.tanh(z), c)
        out = (jnp.dot(f, ws_ref[...], preferred_element_type=f32)
               + bs_ref[...]
               + jnp.dot(h, wn_ref[...].astype(bf), preferred_element_type=f32))
        if relu:
            out = jnp.maximum(out, 0.0)
        if gru is not None:
            hp = hprev_ref[...]
            dh = hp.shape[1]
            og = out[:, :dh]
            gi = (jnp.dot(og, wih_ref[...], preferred_element_type=f32)
                  + bih_ref[...])
            gh = (jnp.dot(hp, whh_ref[...], preferred_element_type=f32)
                  + bhh_ref[...])
            r = sig(gi[:, :dh] + gh[:, :dh])
            zz = sig(gi[:, dh:2 * dh] + gh[:, dh:2 * dh])
            nw = jnp.tanh(gi[:, 2 * dh:] + r * gh[:, 2 * dh:])
            og = (1.0 - zz) * nw + zz * hp
            out = jnp.concatenate(
                [og, jnp.zeros((block, _LANES - dh), f32)], axis=1)
        out_ref[...] = out

    def full(shape):
        return pl.BlockSpec(shape, lambda i: (0,) * len(shape))

    in_specs = [
        pl.BlockSpec((block, _LANES), lambda i: (i, 0)),
        pl.BlockSpec((deg, block, mw), lambda i: (0, i, 0)),
        full((d_in, 4 * d_in)),
        full((d_in, 4 * d_in)),
        full((1, 4 * d_in)),
        full((d_in, _LANES)),
        full((1, _LANES)),
        full((d_in, _LANES)),
    ]
    pad = ((0, 0), (0, _LANES - d_out))
    gs = jnp.concatenate([jnp.full((d_in,), 0.5, f32),
                          jnp.full((d_in,), 0.5, f32),
                          jnp.ones((d_in,), f32),
                          jnp.full((d_in,), 0.5, f32)])
    args = [feats, m_t, p["Wx"] * gs, p["Wh"] * gs,
            (p["b_lstm"] * gs).reshape(1, -1),
            jnp.pad(p["W_self"], pad),
            jnp.pad(p["b_self"].reshape(1, -1), ((0, 0), (0, _LANES - d_out))),
            jnp.pad(p["W_neigh"], pad)]
    if gru is not None:
        hprev, gp = gru
        dh = hprev.shape[1]
        in_specs += [
            pl.BlockSpec((block, dh), lambda i: (i, 0)),
            full((dh, 3 * dh)),
            full((dh, 3 * dh)),
            full((1, 3 * dh)),
            full((1, 3 * dh)),
        ]
        args += [hprev, gp["W_ih"], gp["W_hh"], gp["b_ih"].reshape(1, -1),
                 gp["b_hh"].reshape(1, -1)]

    return pl.pallas_call(
        body,
        grid=(n // block,),
        in_specs=in_specs,
        out_specs=pl.BlockSpec((block, _LANES), lambda i: (i, 0)),
        out_shape=jax.ShapeDtypeStruct((n, _LANES), f32),
        compiler_params=pltpu.CompilerParams(
            dimension_semantics=("arbitrary",)),
    )(*args)


def kernel(x, edge_index, prev_hidden, params):
    n, d_in = x.shape
    e = edge_index.shape[1]
    deg = e // n
    idx_t = edge_index[0].reshape(n, deg).T.reshape(-1)

    m1 = _sc_gather(x, idx_t, 40).reshape(deg, n, d_in)
    h1 = _sage_layer(x, m1, params["conv1"], gru=None, relu=True, block=2000)

    m2 = _sc_gather(h1, idx_t, 40).reshape(deg, n, -1)
    h2 = _sage_layer(h1, m2, params["conv2"], gru=None, relu=True, block=2000)

    m3 = _sc_gather(h2, idx_t, 40).reshape(deg, n, -1)
    h3 = _sage_layer(h2, m3, params["conv3"],
                     gru=(prev_hidden, params["gru"]), relu=True, block=2000)

    m4 = _sc_gather(h3, idx_t, 40).reshape(deg, n, -1)
    logits = _sage_layer(h3, m4, params["conv4"], gru=None, relu=False,
                         block=2000)
    d_out = params["conv4"]["W_self"].shape[1]
    return logits[:, :d_out]


# speedup vs baseline: 1.0079x; 1.0079x over previous
import functools

import jax
import jax.numpy as jnp
from jax import lax
from jax.experimental import pallas as pl
from jax.experimental.pallas import tpu as pltpu
from jax.experimental.pallas import tpu_sc as plsc

_LANES = 128


def _sc_gather(table, idx, chunk):
    _, d = table.shape
    b = idx.shape[0]
    info = plsc.get_sparse_core_info()
    nc = info.num_cores
    nw = nc * info.num_subcores
    b_per_w = b // nw
    assert b_per_w * nw == b and b_per_w % chunk == 0 and chunk % 8 == 0
    n_chunks = b_per_w // chunk
    nbuf = 4
    lag = 3
    n_groups = (n_chunks + lag + nbuf - 1) // nbuf
    mesh = plsc.VectorSubcoreMesh(core_axis_name="c", subcore_axis_name="s")

    @functools.partial(
        pl.kernel,
        mesh=mesh,
        out_type=jax.ShapeDtypeStruct((b, d), table.dtype),
        scratch_types=(
            [pltpu.VMEM((b_per_w,), jnp.int32)]
            + [pltpu.VMEM((chunk, d), table.dtype)] * nbuf
            + [pltpu.SemaphoreType.DMA] * (2 * nbuf)
        ),
    )
    def gather_kernel(table_hbm, idx_hbm, out_hbm, idx_v, *bs):
        bufs = bs[:nbuf]
        gsems = bs[nbuf:2 * nbuf]
        wsems = bs[2 * nbuf:]
        wid = lax.axis_index("s") * nc + lax.axis_index("c")
        base = wid * b_per_w
        pltpu.sync_copy(idx_hbm.at[pl.ds(base, b_per_w)], idx_v)

        def gat(ci, k):
            pltpu.async_copy(
                table_hbm.at[idx_v.at[pl.ds(ci * chunk, chunk)]],
                bufs[k], gsems[k])

        def wrb(ci, k):
            pltpu.async_copy(
                bufs[k], out_hbm.at[pl.ds(base + ci * chunk, chunk)],
                wsems[k])

        def wait_g(k):
            pltpu.make_async_copy(
                table_hbm.at[idx_v.at[pl.ds(0, chunk)]],
                bufs[k], gsems[k]).wait()

        def wait_w(k):
            pltpu.make_async_copy(
                bufs[k], out_hbm.at[pl.ds(base, chunk)], wsems[k]).wait()

        def body(j, carry):
            c0 = nbuf * j
            for k in range(nbuf):
                c = c0 + k

                @pl.when(jnp.logical_and(c >= nbuf, c < n_chunks))
                def _():
                    wait_w(k)

                @pl.when(c < n_chunks)
                def _():
                    gat(c, k)

                c2 = c - lag
                k2 = (k + nbuf - lag) % nbuf

                @pl.when(jnp.logical_and(c2 >= 0, c2 < n_chunks))
                def _():
                    wait_g(k2)
                    wrb(c2, k2)

            return carry

        lax.fori_loop(0, n_groups, body, 0)
        for k in range(nbuf):
            wait_w(k)

    return gather_kernel(table, idx)


def _sage_layer(feats, m_t, p, gru, relu, block):
    n = feats.shape[0]
    deg, _, mw = m_t.shape
    d_in, d_out = p["W_self"].shape
    f32 = jnp.float32

    def sig(v):
        return 0.5 * jnp.tanh(0.5 * v) + 0.5

    fuse = mw == 2 * d_in

    def body(*refs):
        if gru is not None:
            (f_ref, m_ref, wx_ref, wh_ref, bl_ref, ws_ref, bs_ref, wn_ref,
             hprev_ref, wih_ref, whh_ref, bih_ref, bhh_ref, out_ref) = refs
        else:
            (f_ref, m_ref, wx_ref, wh_ref, bl_ref, ws_ref, bs_ref, wn_ref,
             out_ref) = refs
        bf = jnp.bfloat16
        f = f_ref[...][:, :d_in]
        bl = bl_ref[...].astype(bf)
        h = jnp.zeros((block, d_in), bf)
        c = jnp.zeros((block, d_in), bf)
        def lstm_update(T, c):
            ti = T[:, :d_in]
            tf = T[:, d_in:2 * d_in]
            tg = T[:, 2 * d_in:3 * d_in]
            to = T[:, 3 * d_in:]
            c = 0.5 * ((tf + 1.0) * c + (ti + 1.0) * tg)
            h = (0.5 * to + 0.5) * jnp.tanh(c)
            return h, c

        if fuse:
            wcat = jnp.concatenate(
                [wx_ref[...], wh_ref[...]], axis=0).astype(bf)
            zero_low = jnp.zeros((block, d_in), bf)
            hu = jnp.zeros((block, mw), bf)
            for t in range(deg):
                u = m_ref[t].astype(bf) + hu
                z = jnp.dot(u, wcat, preferred_element_type=f32
                            ).astype(bf) + bl
                h, c = lstm_update(jnp.tanh(z), c)
                hu = jnp.concatenate([zero_low, h], axis=1)
        else:
            wx = wx_ref[...].astype(bf)
            wh = wh_ref[...].astype(bf)
            for t in range(deg):
                z = (jnp.dot(m_ref[t][:, :d_in].astype(bf), wx,
                             preferred_element_type=f32)
                     + jnp.dot(h, wh, preferred_element_type=f32)
                     ).astype(bf) + bl
                h, c = lstm_update(jnp.tanh(z), c)
        out = (jnp.dot(f, ws_ref[...], preferred_element_type=f32)
               + bs_ref[...]
               + jnp.dot(h, wn_ref[...].astype(bf), preferred_element_type=f32))
        if relu:
            out = jnp.maximum(out, 0.0)
        if gru is not None:
            hp = hprev_ref[...]
            dh = hp.shape[1]
            og = out[:, :dh]
            gi = (jnp.dot(og, wih_ref[...], preferred_element_type=f32)
                  + bih_ref[...])
            gh = (jnp.dot(hp, whh_ref[...], preferred_element_type=f32)
                  + bhh_ref[...])
            r = sig(gi[:, :dh] + gh[:, :dh])
            zz = sig(gi[:, dh:2 * dh] + gh[:, dh:2 * dh])
            nw = jnp.tanh(gi[:, 2 * dh:] + r * gh[:, 2 * dh:])
            og = (1.0 - zz) * nw + zz * hp
            out = jnp.concatenate(
                [og, jnp.zeros((block, _LANES - dh), f32)], axis=1)
        out_ref[...] = out

    def full(shape):
        return pl.BlockSpec(shape, lambda i: (0,) * len(shape))

    in_specs = [
        pl.BlockSpec((block, _LANES), lambda i: (i, 0)),
        pl.BlockSpec((deg, block, mw), lambda i: (0, i, 0)),
        full((d_in, 4 * d_in)),
        full((d_in, 4 * d_in)),
        full((1, 4 * d_in)),
        full((d_in, _LANES)),
        full((1, _LANES)),
        full((d_in, _LANES)),
    ]
    pad = ((0, 0), (0, _LANES - d_out))
    gs = jnp.concatenate([jnp.full((d_in,), 0.5, f32),
                          jnp.full((d_in,), 0.5, f32),
                          jnp.ones((d_in,), f32),
                          jnp.full((d_in,), 0.5, f32)])
    args = [feats, m_t, p["Wx"] * gs, p["Wh"] * gs,
            (p["b_lstm"] * gs).reshape(1, -1),
            jnp.pad(p["W_self"], pad),
            jnp.pad(p["b_self"].reshape(1, -1), ((0, 0), (0, _LANES - d_out))),
            jnp.pad(p["W_neigh"], pad)]
    if gru is not None:
        hprev, gp = gru
        dh = hprev.shape[1]
        in_specs += [
            pl.BlockSpec((block, dh), lambda i: (i, 0)),
            full((dh, 3 * dh)),
            full((dh, 3 * dh)),
            full((1, 3 * dh)),
            full((1, 3 * dh)),
        ]
        args += [hprev, gp["W_ih"], gp["W_hh"], gp["b_ih"].reshape(1, -1),
                 gp["b_hh"].reshape(1, -1)]

    return pl.pallas_call(
        body,
        grid=(n // block,),
        in_specs=in_specs,
        out_specs=pl.BlockSpec((block, _LANES), lambda i: (i, 0)),
        out_shape=jax.ShapeDtypeStruct((n, _LANES), f32),
        compiler_params=pltpu.CompilerParams(
            dimension_semantics=("arbitrary",)),
    )(*args)


def kernel(x, edge_index, prev_hidden, params):
    n, d_in = x.shape
    e = edge_index.shape[1]
    deg = e // n
    idx_t = edge_index[0].reshape(n, deg).T.reshape(-1)

    m1 = _sc_gather(x, idx_t, 200).reshape(deg, n, d_in)
    h1 = _sage_layer(x, m1, params["conv1"], gru=None, relu=True, block=2000)

    m2 = _sc_gather(h1, idx_t, 200).reshape(deg, n, -1)
    h2 = _sage_layer(h1, m2, params["conv2"], gru=None, relu=True, block=2000)

    m3 = _sc_gather(h2, idx_t, 200).reshape(deg, n, -1)
    h3 = _sage_layer(h2, m3, params["conv3"],
                     gru=(prev_hidden, params["gru"]), relu=True, block=2000)

    m4 = _sc_gather(h3, idx_t, 200).reshape(deg, n, -1)
    logits = _sage_layer(h3, m4, params["conv4"], gru=None, relu=False,
                         block=2000)
    d_out = params["conv4"]["W_self"].shape[1]
    return logits[:, :d_out]
